# baseline (speedup 1.0000x reference)
import functools

import jax
import jax.numpy as jnp
from jax import lax
from jax.experimental import pallas as pl
from jax.experimental.pallas import tpu as pltpu
from jax.experimental.pallas import tpu_sc as plsc

_K = 2
_BLK = 256
_IB = 1024
_TBR = 512
_TBS = 512
_NW = 32


def _pack_bf16(xf32):
    h2 = xf32.shape[1] // 2
    u = lax.bitcast_convert_type(xf32, jnp.int32)
    lsb = jnp.bitwise_and(lax.shift_right_logical(u, 16), 1)
    rb = lax.shift_right_logical(u + 0x7FFF + lsb, 16)
    return jnp.bitwise_or(rb[:, :h2], lax.shift_left(rb[:, h2:], 16))


def _unpack_bf16(xp):
    lo = lax.bitcast_convert_type(lax.shift_left(xp, 16), jnp.float32)
    hi = lax.bitcast_convert_type(
        jnp.bitwise_and(xp, jnp.int32(-65536)), jnp.float32)
    return jnp.concatenate([lo, hi], axis=1)


def _router(x, W_router):
    T, H = x.shape
    E = W_router.shape[1]
    nb = T // _TBR

    def body(x_ref, wr_ref, pos0_ref, pos1_ref, w0_ref, w1_ref, xp_ref,
             cnt_ref, loss_ref, acc_ref):
        i = pl.program_id(0)
        xv = x_ref[...]
        logits = jnp.dot(xv, wr_ref[...], preferred_element_type=jnp.float32)
        m = jnp.max(logits, axis=-1, keepdims=True)
        ex = jnp.exp(logits - m)
        p = ex / jnp.sum(ex, axis=-1, keepdims=True)
        iota = lax.broadcasted_iota(jnp.int32, p.shape, 1)
        m1 = jnp.max(p, axis=-1, keepdims=True)
        id0 = jnp.min(jnp.where(p == m1, iota, E), axis=-1, keepdims=True)
        p2 = jnp.where(iota == id0, -1.0, p)
        m2 = jnp.max(p2, axis=-1, keepdims=True)
        id1 = jnp.min(jnp.where(p2 == m2, iota, E), axis=-1, keepdims=True)
        s = m1 + m2
        oh0 = (iota == id0).astype(jnp.float32)
        oh1 = (iota == id1).astype(jnp.float32)
        hits = oh0 + oh1
        lane = lax.broadcasted_iota(jnp.int32, (1, 128), 1)

        def pad128(v):
            return jnp.concatenate([v, jnp.zeros((1, 128 - E), jnp.float32)],
                                   axis=1)

        @pl.when(i == 0)
        def _():
            acc_ref[...] = jnp.zeros_like(acc_ref)

        w0_ref[...] = m1 / s
        w1_ref[...] = m2 / s
        xp_ref[...] = _pack_bf16(xv)

        @pl.when(i < nb)
        def _():
            acc_ref[0:1, :] += pad128(jnp.sum(p, axis=0, keepdims=True))
            acc_ref[1:2, :] += pad128(jnp.sum(hits, axis=0, keepdims=True))

        @pl.when(i == nb - 1)
        def _():
            loss_ref[0, 0] = (jnp.sum(acc_ref[0:1, :] * acc_ref[1:2, :])
                              * E / (T * T))
            cnt = acc_ref[1:2, :]
            cnt_ref[...] = lax.slice(cnt, (0, 0), (1, E))
            padded = jnp.ceil(cnt / _BLK) * _BLK
            r_l = lax.broadcasted_iota(jnp.int32, (128, 128), 0)
            c_l = lax.broadcasted_iota(jnp.int32, (128, 128), 1)
            excl = (r_l < c_l).astype(jnp.float32)
            acc_ref[2:3, :] = jnp.dot(padded, excl,
                                      preferred_element_type=jnp.float32)
            acc_ref[3:4, :] = jnp.zeros((1, 128), jnp.float32)

        r_t = lax.broadcasted_iota(jnp.int32, (_TBR, _TBR), 0)
        c_t = lax.broadcasted_iota(jnp.int32, (_TBR, _TBR), 1)
        stri = (c_t < r_t).astype(jnp.float32)
        prior = jnp.dot(stri, hits, preferred_element_type=jnp.float32)
        base128 = acc_ref[2:3, :] + acc_ref[3:4, :]
        b8 = lax.slice(base128, (0, 0), (1, E))
        pos0 = jnp.sum((b8 + prior) * oh0, axis=-1, keepdims=True)
        pos1 = jnp.sum((b8 + prior) * oh1, axis=-1, keepdims=True)
        pos0_ref[...] = pos0.astype(jnp.int32)
        pos1_ref[...] = pos1.astype(jnp.int32)

        @pl.when(i >= nb)
        def _():
            acc_ref[3:4, :] += pad128(jnp.sum(hits, axis=0, keepdims=True))

    return pl.pallas_call(
        body,
        grid=(2 * nb,),
        in_specs=[
            pl.BlockSpec((_TBR, H), lambda i: (i % nb, 0)),
            pl.BlockSpec((H, E), lambda i: (0, 0)),
        ],
        out_specs=[
            pl.BlockSpec((_TBR, 1), lambda i: (i % nb, 0)),
            pl.BlockSpec((_TBR, 1), lambda i: (i % nb, 0)),
            pl.BlockSpec((_TBR, 1), lambda i: (i % nb, 0)),
            pl.BlockSpec((_TBR, 1), lambda i: (i % nb, 0)),
            pl.BlockSpec((_TBR, H // 2), lambda i: (i % nb, 0)),
            pl.BlockSpec((1, E), lambda i: (0, 0)),
            pl.BlockSpec((1, 1), lambda i: (0, 0), memory_space=pltpu.SMEM),
        ],
        out_shape=[
            jax.ShapeDtypeStruct((T, 1), jnp.int32),
            jax.ShapeDtypeStruct((T, 1), jnp.int32),
            jax.ShapeDtypeStruct((T, 1), jnp.float32),
            jax.ShapeDtypeStruct((T, 1), jnp.float32),
            jax.ShapeDtypeStruct((T, H // 2), jnp.int32),
            jax.ShapeDtypeStruct((1, E), jnp.float32),
            jax.ShapeDtypeStruct((1, 1), jnp.float32),
        ],
        scratch_shapes=[pltpu.VMEM((8, 128), jnp.float32)],
    )(x, W_router)


def _sc_dispatch(x_pack, pos0, pos1, R):
    T, H2 = x_pack.shape
    tpw = T // _NW
    mesh = plsc.VectorSubcoreMesh(core_axis_name="c", subcore_axis_name="s")

    @functools.partial(
        pl.kernel,
        out_type=jax.ShapeDtypeStruct((R, H2), jnp.int32),
        mesh=mesh,
        scratch_types=[
            pltpu.VMEM((tpw,), jnp.int32),
            pltpu.VMEM((tpw,), jnp.int32),
            pltpu.VMEM((tpw, H2), jnp.int32),
            pltpu.SemaphoreType.DMA,
            pltpu.SemaphoreType.DMA,
        ],
    )
    def k(x_hbm, p0_hbm, p1_hbm, out_hbm, p0_v, p1_v, rows_v, s0, s1):
        wid = lax.axis_index("s") * 2 + lax.axis_index("c")
        t0 = wid * tpw
        pltpu.sync_copy(p0_hbm.at[pl.ds(t0, tpw)], p0_v)
        pltpu.sync_copy(p1_hbm.at[pl.ds(t0, tpw)], p1_v)
        pltpu.sync_copy(x_hbm.at[pl.ds(t0, tpw)], rows_v)
        h0 = pltpu.async_copy(rows_v, out_hbm.at[p0_v], s0)
        h1 = pltpu.async_copy(rows_v, out_hbm.at[p1_v], s1)
        h0.wait()
        h1.wait()

    return k(x_pack, pos0, pos1)


def _sc_gather_rows(table, idx):
    R = idx.shape[0]
    H = table.shape[1]
    per = R // _NW
    isz = 4
    nbuf = 4
    ch = next(c for c in (64, 40, 32, 16, 8)
              if per % c == 0 and nbuf * c * H * isz <= 440_000)
    nch = per // ch
    mesh = plsc.VectorSubcoreMesh(core_axis_name="c", subcore_axis_name="s")

    @functools.partial(
        pl.kernel,
        out_type=jax.ShapeDtypeStruct((R, H), jnp.int32),
        mesh=mesh,
        scratch_types=(
            [pltpu.VMEM((per,), jnp.int32),
             pltpu.VMEM((nbuf, ch, H), jnp.int32)]
            + [pltpu.SemaphoreType.DMA] * (2 * nbuf)
        ),
    )
    def k(idx_hbm, tab_hbm, out_hbm, idx_v, rows_v, *sems):
        gsems = sems[:nbuf]
        wsems = sems[nbuf:]
        wid = lax.axis_index("s") * 2 + lax.axis_index("c")
        base = wid * per
        pltpu.sync_copy(idx_hbm.at[pl.ds(base, per)], idx_v)
        gh = [None] * nbuf
        wh = [None] * nbuf

        def start_gather(c):
            b = c % nbuf
            gh[b] = pltpu.async_copy(
                tab_hbm.at[idx_v.at[pl.ds(c * ch, ch)]], rows_v.at[b],
                gsems[b])

        for c in range(min(nbuf - 1, nch)):
            start_gather(c)
        for c in range(nch):
            b = c % nbuf
            nxt = c + nbuf - 1
            if nxt < nch:
                bb = nxt % nbuf
                if wh[bb] is not None:
                    wh[bb].wait()
                start_gather(nxt)
            gh[b].wait()
            wh[b] = pltpu.async_copy(
                rows_v.at[b], out_hbm.at[pl.ds(base + c * ch, ch)], wsems[b])
        for h in wh:
            if h is not None:
                h.wait()

    return k(idx, table)


def _grouped_ffn(x_sorted, block_expert, W_gate, W_up, W_down):
    R = x_sorted.shape[0]
    H = x_sorted.shape[1] * 2
    E, _, I = W_gate.shape
    G = R // _BLK
    KC = I // _IB

    def body(ids_ref, x_ref, wg_ref, wu_ref, wd_ref, y_ref, acc_ref):
        kc = pl.program_id(1)
        x = _unpack_bf16(x_ref[...])
        g = jnp.dot(x, wg_ref[0], preferred_element_type=jnp.float32)
        u = jnp.dot(x, wu_ref[0], preferred_element_type=jnp.float32)
        a = g * jax.nn.sigmoid(g) * u
        part = jnp.dot(a, wd_ref[0], preferred_element_type=jnp.float32)

        @pl.when(kc == 0)
        def _():
            acc_ref[...] = part

        @pl.when(kc > 0)
        def _():
            acc_ref[...] += part

        @pl.when(kc == KC - 1)
        def _():
            y_ref[...] = _pack_bf16(acc_ref[...])

    grid_spec = pltpu.PrefetchScalarGridSpec(
        num_scalar_prefetch=1,
        grid=(G, KC),
        in_specs=[
            pl.BlockSpec((_BLK, H // 2), lambda g, kc, ids: (g, 0)),
            pl.BlockSpec((1, H, _IB), lambda g, kc, ids: (ids[g], 0, kc)),
            pl.BlockSpec((1, H, _IB), lambda g, kc, ids: (ids[g], 0, kc)),
            pl.BlockSpec((1, _IB, H), lambda g, kc, ids: (ids[g], kc, 0)),
        ],
        out_specs=pl.BlockSpec((_BLK, H // 2), lambda g, kc, ids: (g, 0)),
        scratch_shapes=[pltpu.VMEM((_BLK, H), jnp.float32)],
    )
    return pl.pallas_call(
        body,
        grid_spec=grid_spec,
        out_shape=jax.ShapeDtypeStruct((R, H // 2), jnp.int32),
    )(block_expert, x_sorted, W_gate, W_up, W_down)


def _pair_sum(combined, w0, w1, T):
    H = combined.shape[1] * 2
    nb = T // _TBS

    def body(a_ref, b_ref, g0_ref, g1_ref, o_ref):
        o_ref[...] = (_unpack_bf16(a_ref[...]) * g0_ref[...]
                      + _unpack_bf16(b_ref[...]) * g1_ref[...])

    return pl.pallas_call(
        body,
        grid=(nb,),
        in_specs=[
            pl.BlockSpec((_TBS, H // 2), lambda i: (i, 0)),
            pl.BlockSpec((_TBS, H // 2), lambda i: (i + nb, 0)),
            pl.BlockSpec((_TBS, 1), lambda i: (i, 0)),
            pl.BlockSpec((_TBS, 1), lambda i: (i, 0)),
        ],
        out_specs=pl.BlockSpec((_TBS, H), lambda i: (i, 0)),
        out_shape=jax.ShapeDtypeStruct((T, H), jnp.float32),
    )(combined, combined, w0, w1)


def kernel(hidden_states, W_router, W_gate, W_up, W_down):
    B, S, H = hidden_states.shape
    E = W_router.shape[1]
    T = B * S
    A = _K * T
    G = A // _BLK + E
    R = G * _BLK

    x = hidden_states.reshape(T, H)
    pos0, pos1, w0, w1, x_pack, cnt, loss = _router(x, W_router)

    cnti = cnt[0].astype(jnp.int32)
    nrows_pad = ((cnti + _BLK - 1) // _BLK) * _BLK
    pstart = jnp.concatenate([jnp.zeros((1,), jnp.int32),
                              jnp.cumsum(nrows_pad)])[:E]
    bstart = pstart // _BLK
    block_expert = (jnp.sum(jnp.arange(G)[:, None] >= bstart[None, :], axis=1)
                    .astype(jnp.int32) - 1)
    src = jnp.concatenate([pos0[:, 0], pos1[:, 0]])

    x_sorted = _sc_dispatch(x_pack, pos0[:, 0], pos1[:, 0], R)
    y_pad = _grouped_ffn(x_sorted, block_expert, W_gate, W_up, W_down)
    combined = _sc_gather_rows(y_pad, src)
    out = _pair_sum(combined, w0, w1, T).reshape(B, S, H)
    return out, loss[0, 0]


# speedup vs baseline: 1.3257x; 1.3257x over previous
import functools

import jax
import jax.numpy as jnp
from jax import lax
from jax.experimental import pallas as pl
from jax.experimental.pallas import tpu as pltpu
from jax.experimental.pallas import tpu_sc as plsc

_K = 2
_BLK = 512
_IB = 1024
_TBR = 512
_TBS = 512
_NW = 32


def _pack_bf16(xf32):
    h2 = xf32.shape[1] // 2
    u = lax.bitcast_convert_type(xf32, jnp.int32)
    lsb = jnp.bitwise_and(lax.shift_right_logical(u, 16), 1)
    rb = lax.shift_right_logical(u + 0x7FFF + lsb, 16)
    return jnp.bitwise_or(rb[:, :h2], lax.shift_left(rb[:, h2:], 16))


def _unpack_bf16(xp):
    lo = lax.bitcast_convert_type(lax.shift_left(xp, 16), jnp.float32)
    hi = lax.bitcast_convert_type(
        jnp.bitwise_and(xp, jnp.int32(-65536)), jnp.float32)
    return jnp.concatenate([lo, hi], axis=1)


def _router(x, W_router):
    T, H = x.shape
    E = W_router.shape[1]
    nb = T // _TBR

    def body(x_ref, wr_ref, pos0_ref, pos1_ref, w0_ref, w1_ref, xp_ref,
             cnt_ref, loss_ref, acc_ref):
        i = pl.program_id(0)
        xv = x_ref[...]
        logits = jnp.dot(xv, wr_ref[...], preferred_element_type=jnp.float32)
        m = jnp.max(logits, axis=-1, keepdims=True)
        ex = jnp.exp(logits - m)
        p = ex / jnp.sum(ex, axis=-1, keepdims=True)
        iota = lax.broadcasted_iota(jnp.int32, p.shape, 1)
        m1 = jnp.max(p, axis=-1, keepdims=True)
        id0 = jnp.min(jnp.where(p == m1, iota, E), axis=-1, keepdims=True)
        p2 = jnp.where(iota == id0, -1.0, p)
        m2 = jnp.max(p2, axis=-1, keepdims=True)
        id1 = jnp.min(jnp.where(p2 == m2, iota, E), axis=-1, keepdims=True)
        s = m1 + m2
        oh0 = (iota == id0).astype(jnp.float32)
        oh1 = (iota == id1).astype(jnp.float32)
        hits = oh0 + oh1
        lane = lax.broadcasted_iota(jnp.int32, (1, 128), 1)

        def pad128(v):
            return jnp.concatenate([v, jnp.zeros((1, 128 - E), jnp.float32)],
                                   axis=1)

        @pl.when(i == 0)
        def _():
            acc_ref[...] = jnp.zeros_like(acc_ref)

        w0_ref[...] = m1 / s
        w1_ref[...] = m2 / s
        xp_ref[...] = _pack_bf16(xv)

        @pl.when(i < nb)
        def _():
            acc_ref[0:1, :] += pad128(jnp.sum(p, axis=0, keepdims=True))
            acc_ref[1:2, :] += pad128(jnp.sum(hits, axis=0, keepdims=True))

        @pl.when(i == nb - 1)
        def _():
            loss_ref[0, 0] = (jnp.sum(acc_ref[0:1, :] * acc_ref[1:2, :])
                              * E / (T * T))
            cnt = acc_ref[1:2, :]
            cnt_ref[...] = lax.slice(cnt, (0, 0), (1, E))
            padded = jnp.ceil(cnt / _BLK) * _BLK
            r_l = lax.broadcasted_iota(jnp.int32, (128, 128), 0)
            c_l = lax.broadcasted_iota(jnp.int32, (128, 128), 1)
            excl = (r_l < c_l).astype(jnp.float32)
            acc_ref[2:3, :] = jnp.dot(padded, excl,
                                      preferred_element_type=jnp.float32)
            acc_ref[3:4, :] = jnp.zeros((1, 128), jnp.float32)

        r_t = lax.broadcasted_iota(jnp.int32, (_TBR, _TBR), 0)
        c_t = lax.broadcasted_iota(jnp.int32, (_TBR, _TBR), 1)
        stri = (c_t < r_t).astype(jnp.float32)
        prior = jnp.dot(stri, hits, preferred_element_type=jnp.float32)
        base128 = acc_ref[2:3, :] + acc_ref[3:4, :]
        b8 = lax.slice(base128, (0, 0), (1, E))
        pos0 = jnp.sum((b8 + prior) * oh0, axis=-1, keepdims=True)
        pos1 = jnp.sum((b8 + prior) * oh1, axis=-1, keepdims=True)
        pos0_ref[...] = pos0.astype(jnp.int32)
        pos1_ref[...] = pos1.astype(jnp.int32)

        @pl.when(i >= nb)
        def _():
            acc_ref[3:4, :] += pad128(jnp.sum(hits, axis=0, keepdims=True))

    return pl.pallas_call(
        body,
        grid=(2 * nb,),
        in_specs=[
            pl.BlockSpec((_TBR, H), lambda i: (i % nb, 0)),
            pl.BlockSpec((H, E), lambda i: (0, 0)),
        ],
        out_specs=[
            pl.BlockSpec((_TBR, 1), lambda i: (i % nb, 0)),
            pl.BlockSpec((_TBR, 1), lambda i: (i % nb, 0)),
            pl.BlockSpec((_TBR, 1), lambda i: (i % nb, 0)),
            pl.BlockSpec((_TBR, 1), lambda i: (i % nb, 0)),
            pl.BlockSpec((_TBR, H // 2), lambda i: (i % nb, 0)),
            pl.BlockSpec((1, E), lambda i: (0, 0)),
            pl.BlockSpec((1, 1), lambda i: (0, 0), memory_space=pltpu.SMEM),
        ],
        out_shape=[
            jax.ShapeDtypeStruct((T, 1), jnp.int32),
            jax.ShapeDtypeStruct((T, 1), jnp.int32),
            jax.ShapeDtypeStruct((T, 1), jnp.float32),
            jax.ShapeDtypeStruct((T, 1), jnp.float32),
            jax.ShapeDtypeStruct((T, H // 2), jnp.int32),
            jax.ShapeDtypeStruct((1, E), jnp.float32),
            jax.ShapeDtypeStruct((1, 1), jnp.float32),
        ],
        scratch_shapes=[pltpu.VMEM((8, 128), jnp.float32)],
    )(x, W_router)


def _sc_dispatch(x_pack, pos0, pos1, R):
    T, H2 = x_pack.shape
    tpw = T // _NW
    mesh = plsc.VectorSubcoreMesh(core_axis_name="c", subcore_axis_name="s")

    @functools.partial(
        pl.kernel,
        out_type=jax.ShapeDtypeStruct((R, H2), jnp.int32),
        mesh=mesh,
        scratch_types=[
            pltpu.VMEM((tpw,), jnp.int32),
            pltpu.VMEM((tpw,), jnp.int32),
            pltpu.VMEM((tpw, H2), jnp.int32),
            pltpu.SemaphoreType.DMA,
            pltpu.SemaphoreType.DMA,
        ],
    )
    def k(x_hbm, p0_hbm, p1_hbm, out_hbm, p0_v, p1_v, rows_v, s0, s1):
        wid = lax.axis_index("s") * 2 + lax.axis_index("c")
        t0 = wid * tpw
        pltpu.sync_copy(p0_hbm.at[pl.ds(t0, tpw)], p0_v)
        pltpu.sync_copy(p1_hbm.at[pl.ds(t0, tpw)], p1_v)
        pltpu.sync_copy(x_hbm.at[pl.ds(t0, tpw)], rows_v)
        h0 = pltpu.async_copy(rows_v, out_hbm.at[p0_v], s0)
        h1 = pltpu.async_copy(rows_v, out_hbm.at[p1_v], s1)
        h0.wait()
        h1.wait()

    return k(x_pack, pos0, pos1)


def _sc_gather_rows(table, idx):
    R = idx.shape[0]
    H = table.shape[1]
    per = R // _NW
    isz = 4
    nbuf = 4
    ch = next(c for c in (64, 40, 32, 16, 8)
              if per % c == 0 and nbuf * c * H * isz <= 440_000)
    nch = per // ch
    mesh = plsc.VectorSubcoreMesh(core_axis_name="c", subcore_axis_name="s")

    @functools.partial(
        pl.kernel,
        out_type=jax.ShapeDtypeStruct((R, H), jnp.int32),
        mesh=mesh,
        scratch_types=(
            [pltpu.VMEM((per,), jnp.int32),
             pltpu.VMEM((nbuf, ch, H), jnp.int32)]
            + [pltpu.SemaphoreType.DMA] * (2 * nbuf)
        ),
    )
    def k(idx_hbm, tab_hbm, out_hbm, idx_v, rows_v, *sems):
        gsems = sems[:nbuf]
        wsems = sems[nbuf:]
        wid = lax.axis_index("s") * 2 + lax.axis_index("c")
        base = wid * per
        pltpu.sync_copy(idx_hbm.at[pl.ds(base, per)], idx_v)
        gh = [None] * nbuf
        wh = [None] * nbuf

        def start_gather(c):
            b = c % nbuf
            gh[b] = pltpu.async_copy(
                tab_hbm.at[idx_v.at[pl.ds(c * ch, ch)]], rows_v.at[b],
                gsems[b])

        for c in range(min(nbuf - 1, nch)):
            start_gather(c)
        for c in range(nch):
            b = c % nbuf
            nxt = c + nbuf - 1
            if nxt < nch:
                bb = nxt % nbuf
                if wh[bb] is not None:
                    wh[bb].wait()
                start_gather(nxt)
            gh[b].wait()
            wh[b] = pltpu.async_copy(
                rows_v.at[b], out_hbm.at[pl.ds(base + c * ch, ch)], wsems[b])
        for h in wh:
            if h is not None:
                h.wait()

    return k(idx, table)


def _grouped_ffn(x_sorted, block_expert, W_gate, W_up, W_down):
    R = x_sorted.shape[0]
    H = x_sorted.shape[1] * 2
    E, _, I = W_gate.shape
    G = R // _BLK
    KC = I // _IB

    def body(ids_ref, x_ref, wg_ref, wu_ref, wd_ref, y_ref, acc_ref):
        kc = pl.program_id(1)
        x = _unpack_bf16(x_ref[...])
        g = jnp.dot(x, wg_ref[0], preferred_element_type=jnp.float32)
        u = jnp.dot(x, wu_ref[0], preferred_element_type=jnp.float32)
        a = g * jax.nn.sigmoid(g) * u
        part = jnp.dot(a, wd_ref[0], preferred_element_type=jnp.float32)

        @pl.when(kc == 0)
        def _():
            acc_ref[...] = part

        @pl.when(kc > 0)
        def _():
            acc_ref[...] += part

        @pl.when(kc == KC - 1)
        def _():
            y_ref[...] = _pack_bf16(acc_ref[...])

    grid_spec = pltpu.PrefetchScalarGridSpec(
        num_scalar_prefetch=1,
        grid=(G, KC),
        in_specs=[
            pl.BlockSpec((_BLK, H // 2), lambda g, kc, ids: (g, 0)),
            pl.BlockSpec((1, H, _IB), lambda g, kc, ids: (ids[g], 0, kc)),
            pl.BlockSpec((1, H, _IB), lambda g, kc, ids: (ids[g], 0, kc)),
            pl.BlockSpec((1, _IB, H), lambda g, kc, ids: (ids[g], kc, 0)),
        ],
        out_specs=pl.BlockSpec((_BLK, H // 2), lambda g, kc, ids: (g, 0)),
        scratch_shapes=[pltpu.VMEM((_BLK, H), jnp.float32)],
    )
    return pl.pallas_call(
        body,
        grid_spec=grid_spec,
        out_shape=jax.ShapeDtypeStruct((R, H // 2), jnp.int32),
    )(block_expert, x_sorted, W_gate, W_up, W_down)


def _pair_sum(combined, w0, w1, T):
    H = combined.shape[1] * 2
    nb = T // _TBS

    def body(a_ref, b_ref, g0_ref, g1_ref, o_ref):
        o_ref[...] = (_unpack_bf16(a_ref[...]) * g0_ref[...]
                      + _unpack_bf16(b_ref[...]) * g1_ref[...])

    return pl.pallas_call(
        body,
        grid=(nb,),
        in_specs=[
            pl.BlockSpec((_TBS, H // 2), lambda i: (i, 0)),
            pl.BlockSpec((_TBS, H // 2), lambda i: (i + nb, 0)),
            pl.BlockSpec((_TBS, 1), lambda i: (i, 0)),
            pl.BlockSpec((_TBS, 1), lambda i: (i, 0)),
        ],
        out_specs=pl.BlockSpec((_TBS, H), lambda i: (i, 0)),
        out_shape=jax.ShapeDtypeStruct((T, H), jnp.float32),
    )(combined, combined, w0, w1)


def kernel(hidden_states, W_router, W_gate, W_up, W_down):
    B, S, H = hidden_states.shape
    E = W_router.shape[1]
    T = B * S
    A = _K * T
    G = A // _BLK + E
    R = G * _BLK

    x = hidden_states.reshape(T, H)
    pos0, pos1, w0, w1, x_pack, cnt, loss = _router(x, W_router)

    cnti = cnt[0].astype(jnp.int32)
    nrows_pad = ((cnti + _BLK - 1) // _BLK) * _BLK
    pstart = jnp.concatenate([jnp.zeros((1,), jnp.int32),
                              jnp.cumsum(nrows_pad)])[:E]
    bstart = pstart // _BLK
    block_expert = (jnp.sum(jnp.arange(G)[:, None] >= bstart[None, :], axis=1)
                    .astype(jnp.int32) - 1)
    src = jnp.concatenate([pos0[:, 0], pos1[:, 0]])

    x_sorted = _sc_dispatch(x_pack, pos0[:, 0], pos1[:, 0], R)
    y_pad = _grouped_ffn(x_sorted, block_expert, W_gate, W_up, W_down)
    combined = _sc_gather_rows(y_pad, src)
    out = _pair_sum(combined, w0, w1, T).reshape(B, S, H)
    return out, loss[0, 0]


# speedup vs baseline: 1.3674x; 1.0314x over previous
import functools

import jax
import jax.numpy as jnp
from jax import lax
from jax.experimental import pallas as pl
from jax.experimental.pallas import tpu as pltpu
from jax.experimental.pallas import tpu_sc as plsc

_K = 2
_BLK = 512
_IB = 1024
_TBR = 512
_TBS = 512
_NW = 32


def _pack_bf16(xf32):
    h2 = xf32.shape[1] // 2
    u = lax.bitcast_convert_type(xf32, jnp.int32)
    lsb = jnp.bitwise_and(lax.shift_right_logical(u, 16), 1)
    rb = lax.shift_right_logical(u + 0x7FFF + lsb, 16)
    return jnp.bitwise_or(rb[:, :h2], lax.shift_left(rb[:, h2:], 16))


def _unpack_bf16(xp):
    lo = lax.bitcast_convert_type(lax.shift_left(xp, 16), jnp.float32)
    hi = lax.bitcast_convert_type(
        jnp.bitwise_and(xp, jnp.int32(-65536)), jnp.float32)
    return jnp.concatenate([lo, hi], axis=1)


def _router(x, W_router):
    T, H = x.shape
    E = W_router.shape[1]
    nb = T // _TBR

    def body(x_ref, wr_ref, pos0_ref, pos1_ref, w0_ref, w1_ref, xp_ref,
             cnt_ref, loss_ref, acc_ref):
        i = pl.program_id(0)
        xv = x_ref[...]
        logits = jnp.dot(xv, wr_ref[...], preferred_element_type=jnp.float32)
        m = jnp.max(logits, axis=-1, keepdims=True)
        ex = jnp.exp(logits - m)
        p = ex / jnp.sum(ex, axis=-1, keepdims=True)
        iota = lax.broadcasted_iota(jnp.int32, p.shape, 1)
        m1 = jnp.max(p, axis=-1, keepdims=True)
        id0 = jnp.min(jnp.where(p == m1, iota, E), axis=-1, keepdims=True)
        p2 = jnp.where(iota == id0, -1.0, p)
        m2 = jnp.max(p2, axis=-1, keepdims=True)
        id1 = jnp.min(jnp.where(p2 == m2, iota, E), axis=-1, keepdims=True)
        s = m1 + m2
        oh0 = (iota == id0).astype(jnp.float32)
        oh1 = (iota == id1).astype(jnp.float32)
        hits = oh0 + oh1
        lane = lax.broadcasted_iota(jnp.int32, (1, 128), 1)

        def pad128(v):
            return jnp.concatenate([v, jnp.zeros((1, 128 - E), jnp.float32)],
                                   axis=1)

        @pl.when(i == 0)
        def _():
            acc_ref[...] = jnp.zeros_like(acc_ref)

        w0_ref[...] = m1 / s
        w1_ref[...] = m2 / s
        xp_ref[...] = _pack_bf16(xv)

        @pl.when(i < nb)
        def _():
            acc_ref[0:1, :] += pad128(jnp.sum(p, axis=0, keepdims=True))
            acc_ref[1:2, :] += pad128(jnp.sum(hits, axis=0, keepdims=True))

        @pl.when(i == nb - 1)
        def _():
            loss_ref[0, 0] = (jnp.sum(acc_ref[0:1, :] * acc_ref[1:2, :])
                              * E / (T * T))
            cnt = acc_ref[1:2, :]
            cnt_ref[...] = lax.slice(cnt, (0, 0), (1, E))
            padded = jnp.ceil(cnt / _BLK) * _BLK
            r_l = lax.broadcasted_iota(jnp.int32, (128, 128), 0)
            c_l = lax.broadcasted_iota(jnp.int32, (128, 128), 1)
            excl = (r_l < c_l).astype(jnp.float32)
            acc_ref[2:3, :] = jnp.dot(padded, excl,
                                      preferred_element_type=jnp.float32)
            acc_ref[3:4, :] = jnp.zeros((1, 128), jnp.float32)

        r_t = lax.broadcasted_iota(jnp.int32, (_TBR, _TBR), 0)
        c_t = lax.broadcasted_iota(jnp.int32, (_TBR, _TBR), 1)
        stri = (c_t < r_t).astype(jnp.float32)
        prior = jnp.dot(stri, hits, preferred_element_type=jnp.float32)
        base128 = acc_ref[2:3, :] + acc_ref[3:4, :]
        b8 = lax.slice(base128, (0, 0), (1, E))
        pos0 = jnp.sum((b8 + prior) * oh0, axis=-1, keepdims=True)
        pos1 = jnp.sum((b8 + prior) * oh1, axis=-1, keepdims=True)
        pos0_ref[...] = pos0.astype(jnp.int32)
        pos1_ref[...] = pos1.astype(jnp.int32)

        @pl.when(i >= nb)
        def _():
            acc_ref[3:4, :] += pad128(jnp.sum(hits, axis=0, keepdims=True))

    return pl.pallas_call(
        body,
        grid=(2 * nb,),
        in_specs=[
            pl.BlockSpec((_TBR, H), lambda i: (i % nb, 0)),
            pl.BlockSpec((H, E), lambda i: (0, 0)),
        ],
        out_specs=[
            pl.BlockSpec((_TBR, 1), lambda i: (i % nb, 0)),
            pl.BlockSpec((_TBR, 1), lambda i: (i % nb, 0)),
            pl.BlockSpec((_TBR, 1), lambda i: (i % nb, 0)),
            pl.BlockSpec((_TBR, 1), lambda i: (i % nb, 0)),
            pl.BlockSpec((_TBR, H // 2), lambda i: (i % nb, 0)),
            pl.BlockSpec((1, E), lambda i: (0, 0)),
            pl.BlockSpec((1, 1), lambda i: (0, 0), memory_space=pltpu.SMEM),
        ],
        out_shape=[
            jax.ShapeDtypeStruct((T, 1), jnp.int32),
            jax.ShapeDtypeStruct((T, 1), jnp.int32),
            jax.ShapeDtypeStruct((T, 1), jnp.float32),
            jax.ShapeDtypeStruct((T, 1), jnp.float32),
            jax.ShapeDtypeStruct((T, H // 2), jnp.int32),
            jax.ShapeDtypeStruct((1, E), jnp.float32),
            jax.ShapeDtypeStruct((1, 1), jnp.float32),
        ],
        scratch_shapes=[pltpu.VMEM((8, 128), jnp.float32)],
    )(x, W_router)


def _sc_dispatch(x_pack, pos0, pos1, R):
    T, H2 = x_pack.shape
    tpw = T // _NW
    mesh = plsc.VectorSubcoreMesh(core_axis_name="c", subcore_axis_name="s")

    @functools.partial(
        pl.kernel,
        out_type=jax.ShapeDtypeStruct((R, H2), jnp.int32),
        mesh=mesh,
        scratch_types=[
            pltpu.VMEM((tpw,), jnp.int32),
            pltpu.VMEM((tpw,), jnp.int32),
            pltpu.VMEM((tpw, H2), jnp.int32),
            pltpu.SemaphoreType.DMA,
            pltpu.SemaphoreType.DMA,
        ],
    )
    def k(x_hbm, p0_hbm, p1_hbm, out_hbm, p0_v, p1_v, rows_v, s0, s1):
        wid = lax.axis_index("s") * 2 + lax.axis_index("c")
        t0 = wid * tpw
        pltpu.sync_copy(p0_hbm.at[pl.ds(t0, tpw)], p0_v)
        pltpu.sync_copy(p1_hbm.at[pl.ds(t0, tpw)], p1_v)
        pltpu.sync_copy(x_hbm.at[pl.ds(t0, tpw)], rows_v)
        h0 = pltpu.async_copy(rows_v, out_hbm.at[p0_v], s0)
        h1 = pltpu.async_copy(rows_v, out_hbm.at[p1_v], s1)
        h0.wait()
        h1.wait()

    return k(x_pack, pos0, pos1)


def _sc_gather_rows(table, idx):
    R = idx.shape[0]
    H = table.shape[1]
    per = R // _NW
    isz = 4
    nbuf = 4
    ch = next(c for c in (64, 40, 32, 16, 8)
              if per % c == 0 and nbuf * c * H * isz <= 440_000)
    nch = per // ch
    mesh = plsc.VectorSubcoreMesh(core_axis_name="c", subcore_axis_name="s")

    @functools.partial(
        pl.kernel,
        out_type=jax.ShapeDtypeStruct((R, H), jnp.int32),
        mesh=mesh,
        scratch_types=(
            [pltpu.VMEM((per,), jnp.int32),
             pltpu.VMEM((nbuf, ch, H), jnp.int32)]
            + [pltpu.SemaphoreType.DMA] * (2 * nbuf)
        ),
    )
    def k(idx_hbm, tab_hbm, out_hbm, idx_v, rows_v, *sems):
        gsems = sems[:nbuf]
        wsems = sems[nbuf:]
        wid = lax.axis_index("s") * 2 + lax.axis_index("c")
        base = wid * per
        pltpu.sync_copy(idx_hbm.at[pl.ds(base, per)], idx_v)
        gh = [None] * nbuf
        wh = [None] * nbuf

        def start_gather(c):
            b = c % nbuf
            gh[b] = pltpu.async_copy(
                tab_hbm.at[idx_v.at[pl.ds(c * ch, ch)]], rows_v.at[b],
                gsems[b])

        for c in range(min(nbuf - 1, nch)):
            start_gather(c)
        for c in range(nch):
            b = c % nbuf
            nxt = c + nbuf - 1
            if nxt < nch:
                bb = nxt % nbuf
                if wh[bb] is not None:
                    wh[bb].wait()
                start_gather(nxt)
            gh[b].wait()
            wh[b] = pltpu.async_copy(
                rows_v.at[b], out_hbm.at[pl.ds(base + c * ch, ch)], wsems[b])
        for h in wh:
            if h is not None:
                h.wait()

    return k(idx, table)


def _grouped_ffn(x_sorted, block_expert, valid, W_gate, W_up, W_down):
    R = x_sorted.shape[0]
    H = x_sorted.shape[1] * 2
    E, _, I = W_gate.shape
    G = R // _BLK
    KC = I // _IB

    def body(ids_ref, valid_ref, x_ref, wg_ref, wu_ref, wd_ref, y_ref,
             acc_ref):
        gi = pl.program_id(0)
        kc = pl.program_id(1)

        @pl.when(valid_ref[gi] > 0)
        def _():
            x = _unpack_bf16(x_ref[...])
            g = jnp.dot(x, wg_ref[0], preferred_element_type=jnp.float32)
            u = jnp.dot(x, wu_ref[0], preferred_element_type=jnp.float32)
            a = g * jax.nn.sigmoid(g) * u
            part = jnp.dot(a, wd_ref[0], preferred_element_type=jnp.float32)

            @pl.when(kc == 0)
            def _():
                acc_ref[...] = part

            @pl.when(kc > 0)
            def _():
                acc_ref[...] += part

            @pl.when(kc == KC - 1)
            def _():
                y_ref[...] = _pack_bf16(acc_ref[...])

    grid_spec = pltpu.PrefetchScalarGridSpec(
        num_scalar_prefetch=2,
        grid=(G, KC),
        in_specs=[
            pl.BlockSpec((_BLK, H // 2), lambda g, kc, ids, vv: (g, 0)),
            pl.BlockSpec((1, H, _IB), lambda g, kc, ids, vv: (ids[g], 0, kc)),
            pl.BlockSpec((1, H, _IB), lambda g, kc, ids, vv: (ids[g], 0, kc)),
            pl.BlockSpec((1, _IB, H), lambda g, kc, ids, vv: (ids[g], kc, 0)),
        ],
        out_specs=pl.BlockSpec((_BLK, H // 2), lambda g, kc, ids, vv: (g, 0)),
        scratch_shapes=[pltpu.VMEM((_BLK, H), jnp.float32)],
    )
    return pl.pallas_call(
        body,
        grid_spec=grid_spec,
        out_shape=jax.ShapeDtypeStruct((R, H // 2), jnp.int32),
    )(block_expert, valid, x_sorted, W_gate, W_up, W_down)


def _pair_sum(combined, w0, w1, T):
    H = combined.shape[1] * 2
    nb = T // _TBS

    def body(a_ref, b_ref, g0_ref, g1_ref, o_ref):
        o_ref[...] = (_unpack_bf16(a_ref[...]) * g0_ref[...]
                      + _unpack_bf16(b_ref[...]) * g1_ref[...])

    return pl.pallas_call(
        body,
        grid=(nb,),
        in_specs=[
            pl.BlockSpec((_TBS, H // 2), lambda i: (i, 0)),
            pl.BlockSpec((_TBS, H // 2), lambda i: (i + nb, 0)),
            pl.BlockSpec((_TBS, 1), lambda i: (i, 0)),
            pl.BlockSpec((_TBS, 1), lambda i: (i, 0)),
        ],
        out_specs=pl.BlockSpec((_TBS, H), lambda i: (i, 0)),
        out_shape=jax.ShapeDtypeStruct((T, H), jnp.float32),
    )(combined, combined, w0, w1)


def kernel(hidden_states, W_router, W_gate, W_up, W_down):
    B, S, H = hidden_states.shape
    E = W_router.shape[1]
    T = B * S
    A = _K * T
    G = A // _BLK + E
    R = G * _BLK

    x = hidden_states.reshape(T, H)
    pos0, pos1, w0, w1, x_pack, cnt, loss = _router(x, W_router)

    cnti = cnt[0].astype(jnp.int32)
    nrows_pad = ((cnti + _BLK - 1) // _BLK) * _BLK
    pstart = jnp.concatenate([jnp.zeros((1,), jnp.int32),
                              jnp.cumsum(nrows_pad)])[:E]
    bstart = pstart // _BLK
    block_expert = (jnp.sum(jnp.arange(G)[:, None] >= bstart[None, :], axis=1)
                    .astype(jnp.int32) - 1)
    src = jnp.concatenate([pos0[:, 0], pos1[:, 0]])
    rows_end = pstart + cnti
    valid = (jnp.arange(G, dtype=jnp.int32) * _BLK
             < rows_end[block_expert]).astype(jnp.int32)

    x_sorted = _sc_dispatch(x_pack, pos0[:, 0], pos1[:, 0], R)
    y_pad = _grouped_ffn(x_sorted, block_expert, valid, W_gate, W_up, W_down)
    combined = _sc_gather_rows(y_pad, src)
    out = _pair_sum(combined, w0, w1, T).reshape(B, S, H)
    return out, loss[0, 0]
